# Initial kernel scaffold; baseline (speedup 1.0000x reference)
import jax
import jax.numpy as jnp
from jax.experimental import pallas as pl


def kernel(x, edge_index):
    raise NotImplementedError("write your pallas kernel here")


# speedup vs baseline: 1.4949x; 1.4949x over previous
import functools

import jax
import jax.numpy as jnp
from jax import lax
from jax.experimental import pallas as pl
from jax.experimental.pallas import tpu as pltpu
from jax.experimental.pallas import tpu_sc as plsc

_N_NODES = 10000
_D = 256
_N_EDGES = 160000
_N_GATHER = 2 * _N_EDGES
_CHUNK = 128
_N_CHUNKS = _N_GATHER // _CHUNK
_NW = 32
_BASE_CHUNKS = _N_CHUNKS // _NW
_EXTRA = _N_CHUNKS % _NW


def _sigmoid_body(x_ref, o_ref):
    o_ref[...] = jax.nn.sigmoid(x_ref[...])


def _sigmoid_table(x):
    n, d = x.shape
    blk = 2000
    return pl.pallas_call(
        _sigmoid_body,
        grid=(n // blk,),
        in_specs=[pl.BlockSpec((blk, d), lambda i: (i, 0))],
        out_specs=pl.BlockSpec((blk, d), lambda i: (i, 0)),
        out_shape=jax.ShapeDtypeStruct((n, d), x.dtype),
    )(x)


@functools.partial(
    pl.kernel,
    mesh=plsc.VectorSubcoreMesh(core_axis_name="c", subcore_axis_name="s"),
    out_type=jax.ShapeDtypeStruct((_N_GATHER, _D), jnp.float32),
    scratch_types=[
        pltpu.VMEM((_CHUNK,), jnp.int32),
        pltpu.VMEM((_CHUNK, _D), jnp.float32),
        pltpu.SemaphoreType.DMA,
    ],
)
def _gather_rows(s_hbm, idx_hbm, out_hbm, idx_v, row_v, gsem):
    wid = lax.axis_index("s") * 2 + lax.axis_index("c")
    start = wid * _BASE_CHUNKS + jnp.minimum(wid, _EXTRA)
    count = _BASE_CHUNKS + (wid < _EXTRA).astype(jnp.int32)

    def body(j, carry):
        ch = start + j
        pltpu.sync_copy(idx_hbm.at[ch], idx_v)
        pltpu.async_copy(s_hbm.at[idx_v], row_v, gsem).wait()
        pltpu.sync_copy(row_v, out_hbm.at[pl.ds(ch * _CHUNK, _CHUNK)])
        return carry

    lax.fori_loop(0, count, body, 0)


def kernel(x, edge_index):
    s = _sigmoid_table(x)
    idx = edge_index.astype(jnp.int32)
    combined = idx.T.reshape(_N_GATHER)
    idx2d = combined.reshape(_N_CHUNKS, _CHUNK)
    out = _gather_rows(s, idx2d)
    return out.reshape(_N_EDGES, 2 * _D)


# speedup vs baseline: 1.6911x; 1.1312x over previous
import functools

import jax
import jax.numpy as jnp
from jax import lax
from jax.experimental import pallas as pl
from jax.experimental.pallas import tpu as pltpu
from jax.experimental.pallas import tpu_sc as plsc

_N_NODES = 10000
_D = 256
_N_EDGES = 160000
_N_GATHER = 2 * _N_EDGES
_CHUNK = 128
_N_CHUNKS = _N_GATHER // _CHUNK
_NW = 32
_BASE = _N_CHUNKS // _NW
_EXTRA = _N_CHUNKS % _NW
_MAXC = _BASE + 1
_MAXC_AL = 88


def _sigmoid_body(x_ref, o_ref):
    o_ref[...] = jax.nn.sigmoid(x_ref[...])


def _sigmoid_table(x):
    n, d = x.shape
    blk = 2000
    return pl.pallas_call(
        _sigmoid_body,
        grid=(n // blk,),
        in_specs=[pl.BlockSpec((blk, d), lambda i: (i, 0))],
        out_specs=pl.BlockSpec((blk, d), lambda i: (i, 0)),
        out_shape=jax.ShapeDtypeStruct((n, d), x.dtype),
    )(x)


@functools.partial(
    pl.kernel,
    mesh=plsc.VectorSubcoreMesh(core_axis_name="c", subcore_axis_name="s"),
    out_type=jax.ShapeDtypeStruct((_N_GATHER, _D), jnp.float32),
    scratch_types=[
        pltpu.VMEM((_MAXC_AL, _CHUNK), jnp.int32),
        pltpu.VMEM((_CHUNK, _D), jnp.float32),
        pltpu.VMEM((_CHUNK, _D), jnp.float32),
        pltpu.VMEM((_CHUNK, _D), jnp.float32),
        pltpu.SemaphoreType.DMA,
        pltpu.SemaphoreType.DMA,
        pltpu.SemaphoreType.DMA,
        pltpu.SemaphoreType.DMA,
        pltpu.SemaphoreType.DMA,
        pltpu.SemaphoreType.DMA,
    ],
)
def _gather_rows(s_hbm, idx_hbm, out_hbm, idx_v, b0, b1, b2,
                 g0, g1, g2, o0, o1, o2):
    bufs = (b0, b1, b2)
    gsem = (g0, g1, g2)
    osem = (o0, o1, o2)

    wid = lax.axis_index("s") * 2 + lax.axis_index("c")
    start = wid * _BASE + jnp.minimum(wid, _EXTRA)
    has_extra = wid < _EXTRA
    n = _BASE + has_extra.astype(jnp.int32)

    start_al = (start // 8) * 8
    off = start - start_al
    pltpu.sync_copy(idx_hbm.at[pl.ds(start_al, _MAXC_AL)], idx_v)

    def start_gather(j, b):
        pltpu.async_copy(s_hbm.at[idx_v.at[off + j]], bufs[b], gsem[b])

    def wait_gather(j, b):
        pltpu.make_async_copy(
            s_hbm.at[idx_v.at[off + j]], bufs[b], gsem[b]).wait()

    def start_scatter(j, b):
        pltpu.async_copy(
            bufs[b], out_hbm.at[pl.ds((start + j) * _CHUNK, _CHUNK)], osem[b])

    def wait_scatter(b):
        pltpu.make_async_copy(
            bufs[b], out_hbm.at[pl.ds(0, _CHUNK)], osem[b]).wait()

    for b in range(3):
        start_gather(b, b)

    def triple(t, carry):
        for b in range(3):
            k = 3 * t + b
            wait_gather(k, b)
            start_scatter(k, b)
            bp = (b + 2) % 3

            @pl.when(jnp.logical_and(k >= 1, k + 2 < n))
            def _():
                wait_scatter(bp)
                start_gather(k + 2, bp)

        return carry

    lax.fori_loop(0, _BASE // 3, triple, 0)

    @pl.when(has_extra)
    def _():
        wait_gather(_BASE, _BASE % 3)
        start_scatter(_BASE, _BASE % 3)

    for b in range(3):
        wait_scatter(b)


def kernel(x, edge_index):
    s = _sigmoid_table(x)
    idx = edge_index.astype(jnp.int32)
    combined = idx.T.reshape(_N_GATHER)
    idx2d = combined.reshape(_N_CHUNKS, _CHUNK)
    last_start_al = ((_NW - 1) * _BASE + min(_NW - 1, _EXTRA)) // 8 * 8
    pad = last_start_al + _MAXC_AL - _N_CHUNKS
    idx2d = jnp.concatenate(
        [idx2d, jnp.zeros((pad, _CHUNK), jnp.int32)], axis=0)
    out = _gather_rows(s, idx2d)
    return out.reshape(_N_EDGES, 2 * _D)


# speedup vs baseline: 4.5552x; 2.6937x over previous
import functools

import jax
import jax.numpy as jnp
from jax import lax
from jax.experimental import pallas as pl
from jax.experimental.pallas import tpu as pltpu
from jax.experimental.pallas import tpu_sc as plsc

_N_NODES = 10000
_D = 256
_N_EDGES = 160000
_CHUNK = 64
_N_CHUNKS = _N_EDGES // _CHUNK
_NW = 32
_BASE = _N_CHUNKS // _NW
_EXTRA = _N_CHUNKS % _NW
_IDXCAP = (_BASE + 2) * _CHUNK


def _sigmoid_body(x_ref, o_ref):
    o_ref[...] = jax.nn.sigmoid(x_ref[...])


def _sigmoid_table(x):
    n, d = x.shape
    blk = 2000
    return pl.pallas_call(
        _sigmoid_body,
        grid=(n // blk,),
        in_specs=[pl.BlockSpec((blk, d), lambda i: (i, 0))],
        out_specs=pl.BlockSpec((blk, d), lambda i: (i, 0)),
        out_shape=jax.ShapeDtypeStruct((n, d), x.dtype),
    )(x)


@functools.partial(
    pl.kernel,
    mesh=plsc.VectorSubcoreMesh(core_axis_name="c", subcore_axis_name="s"),
    out_type=jax.ShapeDtypeStruct((_N_EDGES, 2 * _D), jnp.float32),
    scratch_types=[
        pltpu.VMEM((_IDXCAP,), jnp.int32),
        pltpu.VMEM((_IDXCAP,), jnp.int32),
        pltpu.VMEM((_CHUNK, 2 * _D), jnp.float32),
        pltpu.VMEM((_CHUNK, 2 * _D), jnp.float32),
        pltpu.VMEM((_CHUNK, 2 * _D), jnp.float32),
        pltpu.SemaphoreType.DMA,
        pltpu.SemaphoreType.DMA,
        pltpu.SemaphoreType.DMA,
        pltpu.SemaphoreType.DMA,
        pltpu.SemaphoreType.DMA,
        pltpu.SemaphoreType.DMA,
    ],
)
def _gather_rows(s_hbm, edge_hbm, out_hbm, idxs_v, idxd_v, b0, b1, b2,
                 g0, g1, g2, o0, o1, o2):
    bufs = (b0, b1, b2)
    gsem = (g0, g1, g2)
    osem = (o0, o1, o2)

    wid = lax.axis_index("s") * 2 + lax.axis_index("c")
    start = wid * _BASE + jnp.minimum(wid, _EXTRA)
    has_extra = wid < _EXTRA
    n = _BASE + has_extra.astype(jnp.int32)

    start_al = jnp.minimum((start // 2) * 2, (_N_EDGES - _IDXCAP) // _CHUNK)
    off = start - start_al
    pltpu.sync_copy(edge_hbm.at[0, pl.ds(start_al * _CHUNK, _IDXCAP)], idxs_v)
    pltpu.sync_copy(edge_hbm.at[1, pl.ds(start_al * _CHUNK, _IDXCAP)], idxd_v)

    def start_gather(j, b):
        sl = pl.ds((off + j) * _CHUNK, _CHUNK)
        pltpu.async_copy(
            s_hbm.at[idxs_v.at[sl]], bufs[b].at[:, pl.ds(0, _D)], gsem[b])
        pltpu.async_copy(
            s_hbm.at[idxd_v.at[sl]], bufs[b].at[:, pl.ds(_D, _D)], gsem[b])

    def wait_gather(j, b):
        sl = pl.ds((off + j) * _CHUNK, _CHUNK)
        pltpu.make_async_copy(
            s_hbm.at[idxs_v.at[sl]], bufs[b].at[:, pl.ds(0, _D)],
            gsem[b]).wait()
        pltpu.make_async_copy(
            s_hbm.at[idxd_v.at[sl]], bufs[b].at[:, pl.ds(_D, _D)],
            gsem[b]).wait()

    def start_scatter(j, b):
        pltpu.async_copy(
            bufs[b], out_hbm.at[pl.ds((start + j) * _CHUNK, _CHUNK)], osem[b])

    def wait_scatter(b):
        pltpu.make_async_copy(
            bufs[b], out_hbm.at[pl.ds(0, _CHUNK)], osem[b]).wait()

    for b in range(3):
        start_gather(b, b)

    def triple(t, carry):
        for b in range(3):
            k = 3 * t + b
            wait_gather(k, b)
            start_scatter(k, b)
            bp = (b + 2) % 3

            @pl.when(jnp.logical_and(k >= 1, k + 2 < n))
            def _():
                wait_scatter(bp)
                start_gather(k + 2, bp)

        return carry

    lax.fori_loop(0, _BASE // 3, triple, 0)

    @pl.when(has_extra)
    def _():
        wait_gather(_BASE, _BASE % 3)
        start_scatter(_BASE, _BASE % 3)

    for b in range(3):
        wait_scatter(b)


def kernel(x, edge_index):
    s = _sigmoid_table(x)
    return _gather_rows(s, edge_index.astype(jnp.int32))


# speedup vs baseline: 4.6017x; 1.0102x over previous
import functools

import jax
import jax.numpy as jnp
from jax import lax
from jax.experimental import pallas as pl
from jax.experimental.pallas import tpu as pltpu
from jax.experimental.pallas import tpu_sc as plsc

_N_NODES = 10000
_D = 256
_N_EDGES = 160000
_CHUNK = 64
_N_CHUNKS = _N_EDGES // _CHUNK
_NW = 32
_BASE = _N_CHUNKS // _NW
_EXTRA = _N_CHUNKS % _NW
_IDXCAP = (_BASE + 2) * _CHUNK


def _sigmoid_body(x_ref, o_ref):
    o_ref[...] = jax.nn.sigmoid(x_ref[...])


def _sigmoid_table(x):
    n, d = x.shape
    blk = 2000
    return pl.pallas_call(
        _sigmoid_body,
        grid=(n // blk,),
        in_specs=[pl.BlockSpec((blk, d), lambda i: (i, 0))],
        out_specs=pl.BlockSpec((blk, d), lambda i: (i, 0)),
        out_shape=jax.ShapeDtypeStruct((n, d), x.dtype),
    )(x)


@functools.partial(
    pl.kernel,
    mesh=plsc.VectorSubcoreMesh(core_axis_name="c", subcore_axis_name="s"),
    out_type=jax.ShapeDtypeStruct((_N_EDGES, 2 * _D), jnp.float32),
    scratch_types=[
        pltpu.VMEM((_IDXCAP,), jnp.int32),
        pltpu.VMEM((_IDXCAP,), jnp.int32),
        pltpu.VMEM((_CHUNK, 2 * _D), jnp.float32),
        pltpu.VMEM((_CHUNK, 2 * _D), jnp.float32),
        pltpu.VMEM((_CHUNK, 2 * _D), jnp.float32),
        pltpu.SemaphoreType.DMA,
        pltpu.SemaphoreType.DMA,
        pltpu.SemaphoreType.DMA,
        pltpu.SemaphoreType.DMA,
        pltpu.SemaphoreType.DMA,
        pltpu.SemaphoreType.DMA,
    ],
)
def _gather_rows(s_hbm, edge_hbm, out_hbm, idxs_v, idxd_v, b0, b1, b2,
                 g0, g1, g2, o0, o1, o2):
    bufs = (b0, b1, b2)
    gsem = (g0, g1, g2)
    osem = (o0, o1, o2)

    wid = lax.axis_index("s") * 2 + lax.axis_index("c")
    start = wid * _BASE + jnp.minimum(wid, _EXTRA)
    has_extra = wid < _EXTRA
    n = _BASE + has_extra.astype(jnp.int32)

    start_al = jnp.minimum((start // 2) * 2, (_N_EDGES - _IDXCAP) // _CHUNK)
    off = start - start_al
    pltpu.async_copy(
        edge_hbm.at[0, pl.ds(start_al * _CHUNK, _IDXCAP)], idxs_v, g0)
    pltpu.async_copy(
        edge_hbm.at[1, pl.ds(start_al * _CHUNK, _IDXCAP)], idxd_v, g1)
    pltpu.make_async_copy(
        edge_hbm.at[0, pl.ds(start_al * _CHUNK, _IDXCAP)], idxs_v, g0).wait()
    pltpu.make_async_copy(
        edge_hbm.at[1, pl.ds(start_al * _CHUNK, _IDXCAP)], idxd_v, g1).wait()

    def start_gather(j, b):
        sl = pl.ds((off + j) * _CHUNK, _CHUNK)
        pltpu.async_copy(
            s_hbm.at[idxs_v.at[sl]], bufs[b].at[:, pl.ds(0, _D)], gsem[b])
        pltpu.async_copy(
            s_hbm.at[idxd_v.at[sl]], bufs[b].at[:, pl.ds(_D, _D)], gsem[b])

    def wait_gather(j, b):
        sl = pl.ds((off + j) * _CHUNK, _CHUNK)
        pltpu.make_async_copy(
            s_hbm.at[idxs_v.at[sl]], bufs[b].at[:, pl.ds(0, _D)],
            gsem[b]).wait()
        pltpu.make_async_copy(
            s_hbm.at[idxd_v.at[sl]], bufs[b].at[:, pl.ds(_D, _D)],
            gsem[b]).wait()

    def start_scatter(j, b):
        pltpu.async_copy(
            bufs[b], out_hbm.at[pl.ds((start + j) * _CHUNK, _CHUNK)], osem[b])

    def wait_scatter(b):
        pltpu.make_async_copy(
            bufs[b], out_hbm.at[pl.ds(0, _CHUNK)], osem[b]).wait()

    for b in range(3):
        start_gather(b, b)

    def triple(t, carry):
        for b in range(3):
            k = 3 * t + b
            bp = (b + 2) % 3

            @pl.when(jnp.logical_and(k >= 1, k + 2 < n))
            def _():
                wait_scatter(bp)
                start_gather(k + 2, bp)

            wait_gather(k, b)
            start_scatter(k, b)

        return carry

    lax.fori_loop(0, _BASE // 3, triple, 0)

    @pl.when(has_extra)
    def _():
        wait_gather(_BASE, _BASE % 3)
        start_scatter(_BASE, _BASE % 3)

    for b in range(3):
        wait_scatter(b)


def kernel(x, edge_index):
    s = _sigmoid_table(x)
    return _gather_rows(s, edge_index.astype(jnp.int32))
